# Initial kernel scaffold; baseline (speedup 1.0000x reference)
import jax
import jax.numpy as jnp
from jax.experimental import pallas as pl


def kernel(queries, keys, k):
    raise NotImplementedError("write your pallas kernel here")


# speedup vs baseline: 4.1805x; 4.1805x over previous
import functools

import jax
import jax.numpy as jnp
from jax import lax
from jax.experimental import pallas as pl
from jax.experimental.pallas import tpu as pltpu
from jax.experimental.pallas import tpu_sc as plsc

Q = 4096
K = 100000
D = 128
KSEL = 16
W = 128
NB = 784
KPAD = NB * W
KT = 2048
NJ = KPAD // KT
BPJ = KT // W
QT = 512
NI = Q // QT

PAD_VAL = 1e30
MASK_VAL = 3e30
IBIG = 2**30


def _dist_kernel(q_ref, kt_ref, d2_ref, mv_ref):
    j = pl.program_id(1)
    q = q_ref[...]
    kt = kt_ref[...]
    dot = lax.dot_general(q, kt, (((1,), (0,)), ((), ())),
                          preferred_element_type=jnp.float32)
    qsq = jnp.sum(q * q, axis=1, keepdims=True)
    ksq = jnp.sum(kt * kt, axis=0, keepdims=True)
    d2 = qsq - 2.0 * dot + ksq
    col = j * KT + lax.broadcasted_iota(jnp.int32, (QT, KT), 1)
    d2 = jnp.where(col >= K, PAD_VAL, d2)
    d2_ref[...] = d2
    mins = [jnp.min(d2[:, s * W:(s + 1) * W], axis=1, keepdims=True)
            for s in range(BPJ)]
    mv_ref[...] = jnp.concatenate(mins, axis=1).reshape(1, QT, BPJ)


def _distances(queries, keys_t):
    return pl.pallas_call(
        _dist_kernel,
        grid=(NI, NJ),
        in_specs=[
            pl.BlockSpec((QT, D), lambda i, j: (i, 0)),
            pl.BlockSpec((D, KT), lambda i, j: (0, j)),
        ],
        out_specs=[
            pl.BlockSpec((QT, KT), lambda i, j: (i, j)),
            pl.BlockSpec((1, QT, BPJ), lambda i, j: (j, i, 0)),
        ],
        out_shape=[
            jax.ShapeDtypeStruct((Q, KPAD), jnp.float32),
            jax.ShapeDtypeStruct((NJ, Q, BPJ), jnp.float32),
        ],
    )(queries, keys_t)


def _select_blocks_kernel(mv_ref, ids_ref):
    i = pl.program_id(0)
    x = mv_ref[...]
    code = (lax.broadcasted_iota(jnp.int32, (NJ, QT, BPJ), 0) * BPJ
            + lax.broadcasted_iota(jnp.int32, (NJ, QT, BPJ), 2))
    qid = i * QT + lax.broadcasted_iota(jnp.int32, (1, QT), 1)
    for t in range(KSEL):
        m = jnp.min(jnp.min(x, axis=2), axis=0, keepdims=True)
        eq = x == m[:, :, None]
        c2 = jnp.min(jnp.where(eq, code, IBIG), axis=2)
        bsel = jnp.min(c2, axis=0, keepdims=True)
        ids_ref[pl.ds(t, 1), :] = qid * NB + bsel
        x = jnp.where(code == bsel[:, :, None], MASK_VAL, x)


def _select_blocks(mv):
    return pl.pallas_call(
        _select_blocks_kernel,
        grid=(NI,),
        in_specs=[pl.BlockSpec((NJ, QT, BPJ), lambda i: (0, i, 0))],
        out_specs=pl.BlockSpec((KSEL, QT), lambda i: (0, i)),
        out_shape=jax.ShapeDtypeStruct((KSEL, Q), jnp.int32),
    )(mv)


NROWS = Q * KSEL
CH = 128


def _gather_rows(d2rows, idx_flat):
    info = plsc.get_sparse_core_info()
    nw = info.num_cores * info.num_subcores
    rpw = NROWS // nw
    nch = rpw // CH

    @functools.partial(
        pl.kernel,
        mesh=plsc.VectorSubcoreMesh(core_axis_name="c", subcore_axis_name="s"),
        out_type=jax.ShapeDtypeStruct((NROWS, W), jnp.float32),
        scratch_types=[
            pltpu.VMEM((CH,), jnp.int32),
            pltpu.VMEM((CH, W), jnp.float32),
            pltpu.SemaphoreType.DMA,
        ],
    )
    def gather_kernel(rows_hbm, idx_hbm, out_hbm, idx_v, rows_v, sem):
        wid = lax.axis_index("s") * info.num_cores + lax.axis_index("c")
        base = wid * rpw
        for c in range(nch):
            off = base + c * CH
            pltpu.sync_copy(idx_hbm.at[pl.ds(off, CH)], idx_v)
            pltpu.async_copy(rows_hbm.at[idx_v], rows_v, sem).wait()
            pltpu.sync_copy(rows_v, out_hbm.at[pl.ds(off, CH)])

    return gather_kernel(d2rows, idx_flat)


def _final_kernel(cand_ref, ids_ref, vals_ref, idx_ref):
    i = pl.program_id(0)
    c = cand_ref[...]
    ids = ids_ref[...]
    qid = i * QT + lax.broadcasted_iota(jnp.int32, (1, QT), 1)
    bid = ids - qid * NB
    code = (lax.broadcasted_iota(jnp.int32, (KSEL, QT, W), 0) * W
            + lax.broadcasted_iota(jnp.int32, (KSEL, QT, W), 2))
    rowj = lax.broadcasted_iota(jnp.int32, (KSEL, QT), 0)
    for t in range(KSEL):
        m = jnp.min(jnp.min(c, axis=2), axis=0, keepdims=True)
        eq = c == m[:, :, None]
        s2 = jnp.min(jnp.where(eq, code, IBIG), axis=2)
        scode = jnp.min(s2, axis=0, keepdims=True)
        jsel = scode // W
        osel = scode - jsel * W
        bsel = jnp.min(jnp.where(rowj == jsel, bid, IBIG),
                       axis=0, keepdims=True)
        vals_ref[pl.ds(t, 1), :] = jnp.sqrt(jnp.maximum(m, 0.0) + 1e-12)
        idx_ref[pl.ds(t, 1), :] = bsel * W + osel
        c = jnp.where(code == scode[:, :, None], MASK_VAL, c)


def _finalize(cand, ids):
    return pl.pallas_call(
        _final_kernel,
        grid=(NI,),
        in_specs=[
            pl.BlockSpec((KSEL, QT, W), lambda i: (0, i, 0)),
            pl.BlockSpec((KSEL, QT), lambda i: (0, i)),
        ],
        out_specs=[
            pl.BlockSpec((KSEL, QT), lambda i: (0, i)),
            pl.BlockSpec((KSEL, QT), lambda i: (0, i)),
        ],
        out_shape=[
            jax.ShapeDtypeStruct((KSEL, Q), jnp.float32),
            jax.ShapeDtypeStruct((KSEL, Q), jnp.int32),
        ],
    )(cand, ids)


def kernel(queries, keys, k):
    del k
    keys_t = keys.T
    keys_t = jnp.pad(keys_t, ((0, 0), (0, KPAD - K)))
    d2, mv = _distances(queries, keys_t)
    ids = _select_blocks(mv)
    cand = _gather_rows(d2.reshape(Q * NB, W), ids.reshape(NROWS))
    vals_t, idx_t = _finalize(cand.reshape(KSEL, Q, W), ids)
    return vals_t.T, idx_t.T


# speedup vs baseline: 6.3863x; 1.5276x over previous
import functools

import jax
import jax.numpy as jnp
from jax import lax
from jax.experimental import pallas as pl
from jax.experimental.pallas import tpu as pltpu
from jax.experimental.pallas import tpu_sc as plsc

Q = 4096
K = 100000
D = 128
KSEL = 16
W = 128
NB = 784
KPAD = NB * W
KT = 2048
NJ = KPAD // KT
BPJ = KT // W
QT = 512
NI = Q // QT

PAD_VAL = 1e30
MASK_VAL = 3e30
IBIG = 2**30


def _dist_kernel(q_ref, kt_ref, d2_ref, mv_ref):
    j = pl.program_id(1)
    q = q_ref[...]
    kt = kt_ref[...]
    dot = lax.dot_general(q, kt, (((1,), (0,)), ((), ())),
                          preferred_element_type=jnp.float32)
    qsq = jnp.sum(q * q, axis=1, keepdims=True)
    ksq = jnp.sum(kt * kt, axis=0, keepdims=True)
    d2 = qsq - 2.0 * dot + ksq
    col = j * KT + lax.broadcasted_iota(jnp.int32, (QT, KT), 1)
    d2 = jnp.where(col >= K, PAD_VAL, d2)
    d2_ref[...] = d2
    mins = [jnp.min(d2[:, s * W:(s + 1) * W], axis=1, keepdims=True)
            for s in range(BPJ)]
    mv_ref[...] = jnp.concatenate(mins, axis=1).T


def _distances(queries, keys_t):
    return pl.pallas_call(
        _dist_kernel,
        grid=(NI, NJ),
        in_specs=[
            pl.BlockSpec((QT, D), lambda i, j: (i, 0)),
            pl.BlockSpec((D, KT), lambda i, j: (0, j)),
        ],
        out_specs=[
            pl.BlockSpec((QT, KT), lambda i, j: (i, j)),
            pl.BlockSpec((BPJ, QT), lambda i, j: (j, i)),
        ],
        out_shape=[
            jax.ShapeDtypeStruct((Q, KPAD), jnp.float32),
            jax.ShapeDtypeStruct((NB, Q), jnp.float32),
        ],
    )(queries, keys_t)


def _select_blocks_kernel(mv_ref, ids_ref):
    i = pl.program_id(0)
    x = mv_ref[...]
    code = lax.broadcasted_iota(jnp.int32, (NB, QT), 0)
    qid = i * QT + lax.broadcasted_iota(jnp.int32, (1, QT), 1)
    for t in range(KSEL):
        m = jnp.min(x, axis=0, keepdims=True)
        eq = x == m
        bsel = jnp.min(jnp.where(eq, code, IBIG), axis=0, keepdims=True)
        ids_ref[pl.ds(t, 1), :] = qid * NB + bsel
        x = jnp.where(code == bsel, MASK_VAL, x)


def _select_blocks(mv):
    return pl.pallas_call(
        _select_blocks_kernel,
        grid=(NI,),
        in_specs=[pl.BlockSpec((NB, QT), lambda i: (0, i))],
        out_specs=pl.BlockSpec((KSEL, QT), lambda i: (0, i)),
        out_shape=jax.ShapeDtypeStruct((KSEL, Q), jnp.int32),
    )(mv)


NROWS = Q * KSEL
CH = 128


def _gather_rows(d2rows, idx_flat):
    info = plsc.get_sparse_core_info()
    nw = info.num_cores * info.num_subcores
    rpw = NROWS // nw
    nch = rpw // CH

    @functools.partial(
        pl.kernel,
        mesh=plsc.VectorSubcoreMesh(core_axis_name="c", subcore_axis_name="s"),
        out_type=jax.ShapeDtypeStruct((NROWS, W), jnp.float32),
        scratch_types=[
            pltpu.VMEM((CH,), jnp.int32),
            pltpu.VMEM((CH, W), jnp.float32),
            pltpu.SemaphoreType.DMA,
        ],
    )
    def gather_kernel(rows_hbm, idx_hbm, out_hbm, idx_v, rows_v, sem):
        wid = lax.axis_index("s") * info.num_cores + lax.axis_index("c")
        base = wid * rpw
        for c in range(nch):
            off = base + c * CH
            pltpu.sync_copy(idx_hbm.at[pl.ds(off, CH)], idx_v)
            pltpu.async_copy(rows_hbm.at[idx_v], rows_v, sem).wait()
            pltpu.sync_copy(rows_v, out_hbm.at[pl.ds(off, CH)])

    return gather_kernel(d2rows, idx_flat)


def _final_kernel(cand_ref, ids_ref, vals_ref, idx_ref):
    i = pl.program_id(0)
    c = cand_ref[...]
    ids = ids_ref[...]
    qid = i * QT + lax.broadcasted_iota(jnp.int32, (1, QT), 1)
    bid = ids - qid * NB
    code = (lax.broadcasted_iota(jnp.int32, (KSEL, QT, W), 0) * W
            + lax.broadcasted_iota(jnp.int32, (KSEL, QT, W), 2))
    rowj = lax.broadcasted_iota(jnp.int32, (KSEL, QT), 0)
    for t in range(KSEL):
        m = jnp.min(jnp.min(c, axis=2), axis=0, keepdims=True)
        eq = c == m[:, :, None]
        s2 = jnp.min(jnp.where(eq, code, IBIG), axis=2)
        scode = jnp.min(s2, axis=0, keepdims=True)
        jsel = scode // W
        osel = scode - jsel * W
        bsel = jnp.min(jnp.where(rowj == jsel, bid, IBIG),
                       axis=0, keepdims=True)
        vals_ref[pl.ds(t, 1), :] = jnp.sqrt(jnp.maximum(m, 0.0) + 1e-12)
        idx_ref[pl.ds(t, 1), :] = bsel * W + osel
        c = jnp.where(code == scode[:, :, None], MASK_VAL, c)


def _finalize(cand, ids):
    return pl.pallas_call(
        _final_kernel,
        grid=(NI,),
        in_specs=[
            pl.BlockSpec((KSEL, QT, W), lambda i: (0, i, 0)),
            pl.BlockSpec((KSEL, QT), lambda i: (0, i)),
        ],
        out_specs=[
            pl.BlockSpec((KSEL, QT), lambda i: (0, i)),
            pl.BlockSpec((KSEL, QT), lambda i: (0, i)),
        ],
        out_shape=[
            jax.ShapeDtypeStruct((KSEL, Q), jnp.float32),
            jax.ShapeDtypeStruct((KSEL, Q), jnp.int32),
        ],
    )(cand, ids)


def kernel(queries, keys, k):
    del k
    keys_t = keys.T
    keys_t = jnp.pad(keys_t, ((0, 0), (0, KPAD - K)))
    d2, mv = _distances(queries, keys_t)
    ids = _select_blocks(mv)
    cand = _gather_rows(d2.reshape(Q * NB, W), ids.reshape(NROWS))
    vals_t, idx_t = _finalize(cand.reshape(KSEL, Q, W), ids)
    return vals_t.T, idx_t.T


# speedup vs baseline: 10.2031x; 1.5977x over previous
import functools

import jax
import jax.numpy as jnp
from jax import lax
from jax.experimental import pallas as pl
from jax.experimental.pallas import tpu as pltpu
from jax.experimental.pallas import tpu_sc as plsc

Q = 4096
K = 100000
D = 128
KSEL = 16
W = 128
NB = 784
KPAD = NB * W
KT = 2048
NJ = KPAD // KT
BPJ = KT // W
QT = 512
NI = Q // QT

PAD_VAL = 1e30
MASK_VAL = 3e30
IBIG = 2**30


def _dist_kernel(q_ref, kt_ref, d2_ref, mv_ref):
    j = pl.program_id(1)
    q = q_ref[...]
    kt = kt_ref[...]
    dot = lax.dot_general(q, kt, (((1,), (0,)), ((), ())),
                          preferred_element_type=jnp.float32)
    qsq = jnp.sum(q * q, axis=1, keepdims=True)
    ksq = jnp.sum(kt * kt, axis=0, keepdims=True)
    d2 = qsq - 2.0 * dot + ksq
    col = j * KT + lax.broadcasted_iota(jnp.int32, (QT, KT), 1)
    d2 = jnp.where(col >= K, PAD_VAL, d2)
    d2_ref[...] = d2.reshape(QT, BPJ, W)
    mins = [jnp.min(d2[:, s * W:(s + 1) * W], axis=1, keepdims=True)
            for s in range(BPJ)]
    mv_ref[...] = jnp.concatenate(mins, axis=1).T


def _distances(queries, keys_t):
    return pl.pallas_call(
        _dist_kernel,
        grid=(NI, NJ),
        in_specs=[
            pl.BlockSpec((QT, D), lambda i, j: (i, 0)),
            pl.BlockSpec((D, KT), lambda i, j: (0, j)),
        ],
        out_specs=[
            pl.BlockSpec((QT, BPJ, W), lambda i, j: (i, j, 0)),
            pl.BlockSpec((BPJ, QT), lambda i, j: (j, i)),
        ],
        out_shape=[
            jax.ShapeDtypeStruct((Q, NB, W), jnp.float32),
            jax.ShapeDtypeStruct((NB, Q), jnp.float32),
        ],
    )(queries, keys_t)


def _select_blocks_kernel(mv_ref, ids_ref):
    i = pl.program_id(0)
    x = mv_ref[...]
    code = lax.broadcasted_iota(jnp.int32, (NB, QT), 0)
    qid = i * QT + lax.broadcasted_iota(jnp.int32, (1, QT), 1)
    for t in range(KSEL):
        m = jnp.min(x, axis=0, keepdims=True)
        eq = x == m
        bsel = jnp.min(jnp.where(eq, code, IBIG), axis=0, keepdims=True)
        ids_ref[pl.ds(t, 1), :] = qid * NB + bsel
        x = jnp.where(code == bsel, MASK_VAL, x)


def _select_blocks(mv):
    return pl.pallas_call(
        _select_blocks_kernel,
        grid=(NI,),
        in_specs=[pl.BlockSpec((NB, QT), lambda i: (0, i))],
        out_specs=pl.BlockSpec((KSEL, QT), lambda i: (0, i)),
        out_shape=jax.ShapeDtypeStruct((KSEL, Q), jnp.int32),
    )(mv)


NROWS = Q * KSEL
CH = 128


def _gather_rows(d2rows, idx_flat):
    info = plsc.get_sparse_core_info()
    nw = info.num_cores * info.num_subcores
    rpw = NROWS // nw
    nch = rpw // CH

    @functools.partial(
        pl.kernel,
        mesh=plsc.VectorSubcoreMesh(core_axis_name="c", subcore_axis_name="s"),
        out_type=jax.ShapeDtypeStruct((NROWS, W), jnp.float32),
        scratch_types=[
            pltpu.VMEM((CH,), jnp.int32),
            pltpu.VMEM((CH, W), jnp.float32),
            pltpu.SemaphoreType.DMA,
        ],
    )
    def gather_kernel(rows_hbm, idx_hbm, out_hbm, idx_v, rows_v, sem):
        wid = lax.axis_index("s") * info.num_cores + lax.axis_index("c")
        base = wid * rpw
        for c in range(nch):
            off = base + c * CH
            pltpu.sync_copy(idx_hbm.at[pl.ds(off, CH)], idx_v)
            pltpu.async_copy(rows_hbm.at[idx_v], rows_v, sem).wait()
            pltpu.sync_copy(rows_v, out_hbm.at[pl.ds(off, CH)])

    return gather_kernel(d2rows, idx_flat)


def _final_kernel(cand_ref, ids_ref, vals_ref, idx_ref):
    i = pl.program_id(0)
    c = cand_ref[...]
    ids = ids_ref[...]
    qid = i * QT + lax.broadcasted_iota(jnp.int32, (1, QT), 1)
    bid = ids - qid * NB
    code = (lax.broadcasted_iota(jnp.int32, (KSEL, QT, W), 0) * W
            + lax.broadcasted_iota(jnp.int32, (KSEL, QT, W), 2))
    rowj = lax.broadcasted_iota(jnp.int32, (KSEL, QT), 0)
    for t in range(KSEL):
        m = jnp.min(jnp.min(c, axis=2), axis=0, keepdims=True)
        eq = c == m[:, :, None]
        s2 = jnp.min(jnp.where(eq, code, IBIG), axis=2)
        scode = jnp.min(s2, axis=0, keepdims=True)
        jsel = scode // W
        osel = scode - jsel * W
        bsel = jnp.min(jnp.where(rowj == jsel, bid, IBIG),
                       axis=0, keepdims=True)
        vals_ref[pl.ds(t, 1), :] = jnp.sqrt(jnp.maximum(m, 0.0) + 1e-12)
        idx_ref[pl.ds(t, 1), :] = bsel * W + osel
        c = jnp.where(code == scode[:, :, None], MASK_VAL, c)


def _finalize(cand, ids):
    return pl.pallas_call(
        _final_kernel,
        grid=(NI,),
        in_specs=[
            pl.BlockSpec((KSEL, QT, W), lambda i: (0, i, 0)),
            pl.BlockSpec((KSEL, QT), lambda i: (0, i)),
        ],
        out_specs=[
            pl.BlockSpec((KSEL, QT), lambda i: (0, i)),
            pl.BlockSpec((KSEL, QT), lambda i: (0, i)),
        ],
        out_shape=[
            jax.ShapeDtypeStruct((KSEL, Q), jnp.float32),
            jax.ShapeDtypeStruct((KSEL, Q), jnp.int32),
        ],
    )(cand, ids)


def kernel(queries, keys, k):
    del k
    keys_t = keys.T
    keys_t = jnp.pad(keys_t, ((0, 0), (0, KPAD - K)))
    d2, mv = _distances(queries, keys_t)
    ids = _select_blocks(mv)
    cand = _gather_rows(d2.reshape(Q * NB, W), ids.reshape(NROWS))
    vals_t, idx_t = _finalize(cand.reshape(KSEL, Q, W), ids)
    return vals_t.T, idx_t.T


# speedup vs baseline: 10.8513x; 1.0635x over previous
import functools

import jax
import jax.numpy as jnp
from jax import lax
from jax.experimental import pallas as pl
from jax.experimental.pallas import tpu as pltpu
from jax.experimental.pallas import tpu_sc as plsc

Q = 4096
K = 100000
D = 128
KSEL = 16
W = 128
NB = 784
KPAD = NB * W
KT = 2048
NJ = KPAD // KT
BPJ = KT // W
QT = 512
NI = Q // QT
QTA = 1024
NIA = Q // QTA

PAD_VAL = 1e30
MASK_VAL = 3e30
IBIG = 2**30


def _dist_kernel(q_ref, kt_ref, d2_ref, mv_ref):
    j = pl.program_id(1)
    q = q_ref[...]
    kt = kt_ref[...]
    dot = lax.dot_general(q, kt, (((1,), (0,)), ((), ())),
                          preferred_element_type=jnp.float32)
    qsq = jnp.sum(q * q, axis=1, keepdims=True)
    ksq = jnp.sum(kt * kt, axis=0, keepdims=True)
    d2 = qsq - 2.0 * dot + ksq
    col = j * KT + lax.broadcasted_iota(jnp.int32, (QTA, KT), 1)
    d2 = jnp.where(col >= K, PAD_VAL, d2)
    d2_ref[...] = d2.reshape(QTA, BPJ, W)
    mins = [jnp.min(d2[:, s * W:(s + 1) * W], axis=1, keepdims=True)
            for s in range(BPJ)]
    mv_ref[...] = jnp.concatenate(mins, axis=1).T


def _distances(queries, keys_t):
    return pl.pallas_call(
        _dist_kernel,
        grid=(NIA, NJ),
        in_specs=[
            pl.BlockSpec((QTA, D), lambda i, j: (i, 0)),
            pl.BlockSpec((D, KT), lambda i, j: (0, j)),
        ],
        out_specs=[
            pl.BlockSpec((QTA, BPJ, W), lambda i, j: (i, j, 0)),
            pl.BlockSpec((BPJ, QTA), lambda i, j: (j, i)),
        ],
        out_shape=[
            jax.ShapeDtypeStruct((Q, NB, W), jnp.float32),
            jax.ShapeDtypeStruct((NB, Q), jnp.float32),
        ],
    )(queries, keys_t)


def _select_blocks_kernel(mv_ref, ids_ref):
    i = pl.program_id(0)
    x = mv_ref[...]
    code = lax.broadcasted_iota(jnp.int32, (NB, QT), 0)
    qid = i * QT + lax.broadcasted_iota(jnp.int32, (1, QT), 1)
    for t in range(KSEL):
        m = jnp.min(x, axis=0, keepdims=True)
        eq = x == m
        bsel = jnp.min(jnp.where(eq, code, IBIG), axis=0, keepdims=True)
        ids_ref[pl.ds(t, 1), :] = qid * NB + bsel
        x = jnp.where(code == bsel, MASK_VAL, x)


def _select_blocks(mv):
    return pl.pallas_call(
        _select_blocks_kernel,
        grid=(NI,),
        in_specs=[pl.BlockSpec((NB, QT), lambda i: (0, i))],
        out_specs=pl.BlockSpec((KSEL, QT), lambda i: (0, i)),
        out_shape=jax.ShapeDtypeStruct((KSEL, Q), jnp.int32),
    )(mv)


NROWS = Q * KSEL
CH = 128


def _gather_rows(d2rows, idx_flat):
    info = plsc.get_sparse_core_info()
    nw = info.num_cores * info.num_subcores
    rpw = NROWS // nw
    nch = rpw // CH

    @functools.partial(
        pl.kernel,
        mesh=plsc.VectorSubcoreMesh(core_axis_name="c", subcore_axis_name="s"),
        out_type=jax.ShapeDtypeStruct((NROWS, W), jnp.float32),
        scratch_types=[
            pltpu.VMEM((CH,), jnp.int32),
            pltpu.VMEM((CH, W), jnp.float32),
            pltpu.SemaphoreType.DMA,
        ],
    )
    def gather_kernel(rows_hbm, idx_hbm, out_hbm, idx_v, rows_v, sem):
        wid = lax.axis_index("s") * info.num_cores + lax.axis_index("c")
        base = wid * rpw
        for c in range(nch):
            off = base + c * CH
            pltpu.sync_copy(idx_hbm.at[pl.ds(off, CH)], idx_v)
            pltpu.async_copy(rows_hbm.at[idx_v], rows_v, sem).wait()
            pltpu.sync_copy(rows_v, out_hbm.at[pl.ds(off, CH)])

    return gather_kernel(d2rows, idx_flat)


def _final_kernel(cand_ref, ids_ref, vals_ref, idx_ref):
    i = pl.program_id(0)
    c = cand_ref[...]
    ids = ids_ref[...]
    qid = i * QT + lax.broadcasted_iota(jnp.int32, (1, QT), 1)
    bid = ids - qid * NB
    code = (lax.broadcasted_iota(jnp.int32, (KSEL, QT, W), 0) * W
            + lax.broadcasted_iota(jnp.int32, (KSEL, QT, W), 2))
    rowj = lax.broadcasted_iota(jnp.int32, (KSEL, QT), 0)
    for t in range(KSEL):
        m = jnp.min(jnp.min(c, axis=2), axis=0, keepdims=True)
        eq = c == m[:, :, None]
        s2 = jnp.min(jnp.where(eq, code, IBIG), axis=2)
        scode = jnp.min(s2, axis=0, keepdims=True)
        jsel = scode // W
        osel = scode - jsel * W
        bsel = jnp.min(jnp.where(rowj == jsel, bid, IBIG),
                       axis=0, keepdims=True)
        vals_ref[pl.ds(t, 1), :] = jnp.sqrt(jnp.maximum(m, 0.0) + 1e-12)
        idx_ref[pl.ds(t, 1), :] = bsel * W + osel
        c = jnp.where(code == scode[:, :, None], MASK_VAL, c)


def _finalize(cand, ids):
    return pl.pallas_call(
        _final_kernel,
        grid=(NI,),
        in_specs=[
            pl.BlockSpec((KSEL, QT, W), lambda i: (0, i, 0)),
            pl.BlockSpec((KSEL, QT), lambda i: (0, i)),
        ],
        out_specs=[
            pl.BlockSpec((KSEL, QT), lambda i: (0, i)),
            pl.BlockSpec((KSEL, QT), lambda i: (0, i)),
        ],
        out_shape=[
            jax.ShapeDtypeStruct((KSEL, Q), jnp.float32),
            jax.ShapeDtypeStruct((KSEL, Q), jnp.int32),
        ],
    )(cand, ids)


def kernel(queries, keys, k):
    del k
    keys_t = keys.T
    keys_t = jnp.pad(keys_t, ((0, 0), (0, KPAD - K)))
    d2, mv = _distances(queries, keys_t)
    ids = _select_blocks(mv)
    cand = _gather_rows(d2.reshape(Q * NB, W), ids.reshape(NROWS))
    vals_t, idx_t = _finalize(cand.reshape(KSEL, Q, W), ids)
    return vals_t.T, idx_t.T


# speedup vs baseline: 11.1435x; 1.0269x over previous
import functools

import jax
import jax.numpy as jnp
from jax import lax
from jax.experimental import pallas as pl
from jax.experimental.pallas import tpu as pltpu
from jax.experimental.pallas import tpu_sc as plsc

Q = 4096
K = 100000
D = 128
KSEL = 16
W = 128
NB = 784
KPAD = NB * W
KT = 2048
NJ = KPAD // KT
BPJ = KT // W
QT = 512
NI = Q // QT
QTA = 2048
NIA = Q // QTA

PAD_VAL = 1e30
MASK_VAL = 3e30
IBIG = 2**30


def _dist_kernel(q_ref, kt_ref, d2_ref, mv_ref):
    j = pl.program_id(1)
    q = q_ref[...]
    kt = kt_ref[...]
    dot = lax.dot_general(q, kt, (((1,), (0,)), ((), ())),
                          preferred_element_type=jnp.float32)
    qsq = jnp.sum(q * q, axis=1, keepdims=True)
    ksq = jnp.sum(kt * kt, axis=0, keepdims=True)
    d2 = qsq - 2.0 * dot + ksq
    col = j * KT + lax.broadcasted_iota(jnp.int32, (QTA, KT), 1)
    d2 = jnp.where(col >= K, PAD_VAL, d2)
    d2_ref[...] = d2.reshape(QTA, BPJ, W)
    mins = [jnp.min(d2[:, s * W:(s + 1) * W], axis=1, keepdims=True)
            for s in range(BPJ)]
    mv_ref[...] = jnp.concatenate(mins, axis=1).T


def _distances(queries, keys_t):
    return pl.pallas_call(
        _dist_kernel,
        grid=(NIA, NJ),
        in_specs=[
            pl.BlockSpec((QTA, D), lambda i, j: (i, 0)),
            pl.BlockSpec((D, KT), lambda i, j: (0, j)),
        ],
        out_specs=[
            pl.BlockSpec((QTA, BPJ, W), lambda i, j: (i, j, 0)),
            pl.BlockSpec((BPJ, QTA), lambda i, j: (j, i)),
        ],
        out_shape=[
            jax.ShapeDtypeStruct((Q, NB, W), jnp.float32),
            jax.ShapeDtypeStruct((NB, Q), jnp.float32),
        ],
    )(queries, keys_t)


def _select_blocks_kernel(mv_ref, ids_ref):
    i = pl.program_id(0)
    x = mv_ref[...]
    code = lax.broadcasted_iota(jnp.int32, (NB, QT), 0)
    qid = i * QT + lax.broadcasted_iota(jnp.int32, (1, QT), 1)
    for t in range(KSEL):
        m = jnp.min(x, axis=0, keepdims=True)
        eq = x == m
        bsel = jnp.min(jnp.where(eq, code, IBIG), axis=0, keepdims=True)
        ids_ref[pl.ds(t, 1), :] = qid * NB + bsel
        x = jnp.where(code == bsel, MASK_VAL, x)


def _select_blocks(mv):
    return pl.pallas_call(
        _select_blocks_kernel,
        grid=(NI,),
        in_specs=[pl.BlockSpec((NB, QT), lambda i: (0, i))],
        out_specs=pl.BlockSpec((KSEL, QT), lambda i: (0, i)),
        out_shape=jax.ShapeDtypeStruct((KSEL, Q), jnp.int32),
    )(mv)


NROWS = Q * KSEL
CH = 128


def _gather_rows(d2rows, idx_flat):
    info = plsc.get_sparse_core_info()
    nw = info.num_cores * info.num_subcores
    rpw = NROWS // nw
    nch = rpw // CH

    @functools.partial(
        pl.kernel,
        mesh=plsc.VectorSubcoreMesh(core_axis_name="c", subcore_axis_name="s"),
        out_type=jax.ShapeDtypeStruct((NROWS, W), jnp.float32),
        scratch_types=[
            pltpu.VMEM((CH,), jnp.int32),
            pltpu.VMEM((CH, W), jnp.float32),
            pltpu.SemaphoreType.DMA,
        ],
    )
    def gather_kernel(rows_hbm, idx_hbm, out_hbm, idx_v, rows_v, sem):
        wid = lax.axis_index("s") * info.num_cores + lax.axis_index("c")
        base = wid * rpw
        for c in range(nch):
            off = base + c * CH
            pltpu.sync_copy(idx_hbm.at[pl.ds(off, CH)], idx_v)
            pltpu.async_copy(rows_hbm.at[idx_v], rows_v, sem).wait()
            pltpu.sync_copy(rows_v, out_hbm.at[pl.ds(off, CH)])

    return gather_kernel(d2rows, idx_flat)


def _final_kernel(cand_ref, ids_ref, vals_ref, idx_ref):
    i = pl.program_id(0)
    c = cand_ref[...]
    ids = ids_ref[...]
    qid = i * QT + lax.broadcasted_iota(jnp.int32, (1, QT), 1)
    bid = ids - qid * NB
    code = (lax.broadcasted_iota(jnp.int32, (KSEL, QT, W), 0) * W
            + lax.broadcasted_iota(jnp.int32, (KSEL, QT, W), 2))
    rowj = lax.broadcasted_iota(jnp.int32, (KSEL, QT), 0)
    for t in range(KSEL):
        m = jnp.min(jnp.min(c, axis=2), axis=0, keepdims=True)
        eq = c == m[:, :, None]
        s2 = jnp.min(jnp.where(eq, code, IBIG), axis=2)
        scode = jnp.min(s2, axis=0, keepdims=True)
        jsel = scode // W
        osel = scode - jsel * W
        bsel = jnp.min(jnp.where(rowj == jsel, bid, IBIG),
                       axis=0, keepdims=True)
        vals_ref[pl.ds(t, 1), :] = jnp.sqrt(jnp.maximum(m, 0.0) + 1e-12)
        idx_ref[pl.ds(t, 1), :] = bsel * W + osel
        c = jnp.where(code == scode[:, :, None], MASK_VAL, c)


def _finalize(cand, ids):
    return pl.pallas_call(
        _final_kernel,
        grid=(NI,),
        in_specs=[
            pl.BlockSpec((KSEL, QT, W), lambda i: (0, i, 0)),
            pl.BlockSpec((KSEL, QT), lambda i: (0, i)),
        ],
        out_specs=[
            pl.BlockSpec((KSEL, QT), lambda i: (0, i)),
            pl.BlockSpec((KSEL, QT), lambda i: (0, i)),
        ],
        out_shape=[
            jax.ShapeDtypeStruct((KSEL, Q), jnp.float32),
            jax.ShapeDtypeStruct((KSEL, Q), jnp.int32),
        ],
    )(cand, ids)


def kernel(queries, keys, k):
    del k
    keys_t = keys.T
    keys_t = jnp.pad(keys_t, ((0, 0), (0, KPAD - K)))
    d2, mv = _distances(queries, keys_t)
    ids = _select_blocks(mv)
    cand = _gather_rows(d2.reshape(Q * NB, W), ids.reshape(NROWS))
    vals_t, idx_t = _finalize(cand.reshape(KSEL, Q, W), ids)
    return vals_t.T, idx_t.T


# speedup vs baseline: 14.6150x; 1.3115x over previous
import functools

import jax
import jax.numpy as jnp
from jax import lax
from jax.experimental import pallas as pl
from jax.experimental.pallas import tpu as pltpu
from jax.experimental.pallas import tpu_sc as plsc

Q = 4096
K = 100000
D = 128
KSEL = 16
W = 128
NB = 784
KPAD = NB * W
KT = 2048
NJ = KPAD // KT
BPJ = KT // W
QT = 512
NI = Q // QT
QTA = 2048
NIA = Q // QTA

PAD_VAL = 1e30
MASK_VAL = 3e30
IBIG = 2**30


def _dist_kernel(q_ref, kt_ref, d2_ref, mv_ref):
    j = pl.program_id(1)
    q = q_ref[...]
    kt = kt_ref[...]
    dot = lax.dot_general(q, kt, (((1,), (0,)), ((), ())),
                          preferred_element_type=jnp.float32)
    qsq = jnp.sum(q * q, axis=1, keepdims=True)
    ksq = jnp.sum(kt * kt, axis=0, keepdims=True)
    d2 = qsq - 2.0 * dot + ksq
    col = j * KT + lax.broadcasted_iota(jnp.int32, (QTA, KT), 1)
    d2 = jnp.where(col >= K, PAD_VAL, d2)
    d2_ref[...] = d2.reshape(QTA, BPJ, W)
    mins = [jnp.min(d2[:, s * W:(s + 1) * W], axis=1, keepdims=True)
            for s in range(BPJ)]
    mv_ref[...] = jnp.concatenate(mins, axis=1).T


def _distances(queries, keys_t):
    return pl.pallas_call(
        _dist_kernel,
        grid=(NIA, NJ),
        in_specs=[
            pl.BlockSpec((QTA, D), lambda i, j: (i, 0)),
            pl.BlockSpec((D, KT), lambda i, j: (0, j)),
        ],
        out_specs=[
            pl.BlockSpec((QTA, BPJ, W), lambda i, j: (i, j, 0)),
            pl.BlockSpec((BPJ, QTA), lambda i, j: (j, i)),
        ],
        out_shape=[
            jax.ShapeDtypeStruct((Q, NB, W), jnp.float32),
            jax.ShapeDtypeStruct((NB, Q), jnp.float32),
        ],
    )(queries, keys_t)


def _select_blocks_kernel(mv_ref, ids_ref):
    i = pl.program_id(0)
    x = mv_ref[...]
    code = lax.broadcasted_iota(jnp.int32, (NB, QT), 0)
    qid = i * QT + lax.broadcasted_iota(jnp.int32, (1, QT), 1)
    for t in range(KSEL):
        m = jnp.min(x, axis=0, keepdims=True)
        eq = x == m
        bsel = jnp.min(jnp.where(eq, code, IBIG), axis=0, keepdims=True)
        ids_ref[pl.ds(t, 1), :] = qid * NB + bsel
        x = jnp.where(code == bsel, MASK_VAL, x)


def _select_blocks(mv):
    return pl.pallas_call(
        _select_blocks_kernel,
        grid=(NI,),
        in_specs=[pl.BlockSpec((NB, QT), lambda i: (0, i))],
        out_specs=pl.BlockSpec((KSEL, QT), lambda i: (0, i)),
        out_shape=jax.ShapeDtypeStruct((KSEL, Q), jnp.int32),
    )(mv)


NROWS = Q * KSEL
CH = 128


def _gather_rows(d2rows, idx_flat):
    info = plsc.get_sparse_core_info()
    nw = info.num_cores * info.num_subcores
    rpw = NROWS // nw
    nch = rpw // CH

    @functools.partial(
        pl.kernel,
        mesh=plsc.VectorSubcoreMesh(core_axis_name="c", subcore_axis_name="s"),
        out_type=jax.ShapeDtypeStruct((NROWS, W), jnp.float32),
        scratch_types=[
            pltpu.VMEM((CH,), jnp.int32),
            pltpu.VMEM((CH, W), jnp.float32),
            pltpu.SemaphoreType.DMA,
        ],
    )
    def gather_kernel(rows_hbm, idx_hbm, out_hbm, idx_v, rows_v, sem):
        wid = lax.axis_index("s") * info.num_cores + lax.axis_index("c")
        base = wid * rpw
        for c in range(nch):
            off = base + c * CH
            pltpu.sync_copy(idx_hbm.at[pl.ds(off, CH)], idx_v)
            pltpu.async_copy(rows_hbm.at[idx_v], rows_v, sem).wait()
            pltpu.sync_copy(rows_v, out_hbm.at[pl.ds(off, CH)])

    return gather_kernel(d2rows, idx_flat)


def _final_kernel(cand_ref, ids_ref, vals_ref, idx_ref):
    i = pl.program_id(0)
    c = cand_ref[...]
    ids = ids_ref[...]
    qid = i * QT + lax.broadcasted_iota(jnp.int32, (1, QT), 1)
    bid = ids - qid * NB
    rowj = lax.broadcasted_iota(jnp.int32, (KSEL, QT), 0)
    rowj3 = lax.broadcasted_iota(jnp.int32, (KSEL, QT, W), 0)
    lane = lax.broadcasted_iota(jnp.int32, (QT, W), 1)
    sm = jnp.min(c, axis=2)
    for t in range(KSEL):
        m = jnp.min(sm, axis=0, keepdims=True)
        jsel = jnp.min(jnp.where(sm == m, rowj, IBIG),
                       axis=0, keepdims=True)
        onehot = rowj == jsel
        oh3 = rowj3 == jsel[:, :, None]
        strip = jnp.max(jnp.where(oh3, c, -1.0), axis=0)
        ms = jnp.min(strip, axis=1, keepdims=True)
        osel = jnp.min(jnp.where(strip == ms, lane, IBIG),
                       axis=1, keepdims=True)
        hit = lane == osel
        newmin = jnp.min(jnp.where(hit, MASK_VAL, strip),
                         axis=1, keepdims=True)
        bsel = jnp.min(jnp.where(onehot, bid, IBIG),
                       axis=0, keepdims=True)
        vals_ref[pl.ds(t, 1), :] = jnp.sqrt(jnp.maximum(m, 0.0) + 1e-12)
        idx_ref[pl.ds(t, 1), :] = bsel * W + osel.T
        sm = jnp.where(onehot, newmin.T, sm)
        c = jnp.where(oh3 & hit[None, :, :], MASK_VAL, c)


def _finalize(cand, ids):
    return pl.pallas_call(
        _final_kernel,
        grid=(NI,),
        in_specs=[
            pl.BlockSpec((KSEL, QT, W), lambda i: (0, i, 0)),
            pl.BlockSpec((KSEL, QT), lambda i: (0, i)),
        ],
        out_specs=[
            pl.BlockSpec((KSEL, QT), lambda i: (0, i)),
            pl.BlockSpec((KSEL, QT), lambda i: (0, i)),
        ],
        out_shape=[
            jax.ShapeDtypeStruct((KSEL, Q), jnp.float32),
            jax.ShapeDtypeStruct((KSEL, Q), jnp.int32),
        ],
    )(cand, ids)


def kernel(queries, keys, k):
    del k
    keys_t = keys.T
    keys_t = jnp.pad(keys_t, ((0, 0), (0, KPAD - K)))
    d2, mv = _distances(queries, keys_t)
    ids = _select_blocks(mv)
    cand = _gather_rows(d2.reshape(Q * NB, W), ids.reshape(NROWS))
    vals_t, idx_t = _finalize(cand.reshape(KSEL, Q, W), ids)
    return vals_t.T, idx_t.T
